# baseline (speedup 1.0000x reference)
import jax
import jax.numpy as jnp
from jax import lax
from jax.experimental import pallas as pl
from jax.experimental.pallas import tpu as pltpu
from jax.experimental.pallas import tpu_sc as plsc

_B = 64
_T = 1024
_N = _B * _T
_NC = 2
_NS = 16
_NW = _NC * _NS
_CHUNK = _N // _NW
_GB = 128
_NG = _CHUNK // _GB
_D = 128
_NB = 4
_NGRP = _NG // _NB


def _sc_body(h_hbm, w_hbm, tab_hbm, sin_hbm, cos_hbm,
             h_v, w_v, idx_v, rows, *sems):
    gsem = sems[:_NB]
    ssem = sems[_NB:2 * _NB]
    csem = sems[2 * _NB:]
    wid = lax.axis_index("s") * _NC + lax.axis_index("c")
    base = wid * _CHUNK
    pltpu.sync_copy(h_hbm.at[pl.ds(base, _CHUNK)], h_v)
    pltpu.sync_copy(w_hbm.at[pl.ds(base, _CHUNK)], w_v)

    def idx_body(j, carry):
        for k in range(_GB // 16):
            t0 = j * _GB + k * 16
            idx_v[j, pl.ds(k * 16, 16)] = h_v[pl.ds(t0, 16)] * 32 + w_v[pl.ds(t0, 16)]
        return carry

    lax.fori_loop(0, _NG, idx_body, 0)

    def fire_gather(j, b):
        pltpu.async_copy(tab_hbm.at[idx_v.at[j]], rows.at[b], gsem[b])

    for b in range(_NB):
        fire_gather(b, b)

    def group_body(g, carry):
        for b in range(_NB):
            j = g * _NB + b
            pltpu.make_async_copy(tab_hbm.at[idx_v.at[j]], rows.at[b],
                                  gsem[b]).wait()
            tok = pl.ds(base + j * _GB, _GB)
            cs = pltpu.async_copy(rows.at[b, slice(None), pl.ds(0, 64)],
                                  sin_hbm.at[tok], ssem[b])
            cc = pltpu.async_copy(rows.at[b, slice(None), pl.ds(64, 64)],
                                  cos_hbm.at[tok], csem[b])

            @pl.when(g < _NGRP - 1)
            def _():
                cs.wait()
                cc.wait()
                fire_gather(j + _NB, b)

        return carry

    lax.fori_loop(0, _NGRP, group_body, 0)

    for b in range(_NB):
        j = (_NGRP - 1) * _NB + b
        tok = pl.ds(base + j * _GB, _GB)
        pltpu.make_async_copy(rows.at[b, slice(None), pl.ds(0, 64)],
                              sin_hbm.at[tok], ssem[b]).wait()
        pltpu.make_async_copy(rows.at[b, slice(None), pl.ds(64, 64)],
                              cos_hbm.at[tok], csem[b]).wait()


@jax.jit
def _rope_sc(grid, cos_h_all, sin_h_all, cos_w_all, sin_w_all):
    h_n, f = cos_h_all.shape
    w_n = cos_w_all.shape[0]
    ch = jnp.broadcast_to(cos_h_all[:, None, :], (h_n, w_n, f))
    cw = jnp.broadcast_to(cos_w_all[None, :, :], (h_n, w_n, f))
    sh = jnp.broadcast_to(sin_h_all[:, None, :], (h_n, w_n, f))
    sw = jnp.broadcast_to(sin_w_all[None, :, :], (h_n, w_n, f))
    tab = jnp.concatenate([sh, sw, sh, sw, ch, cw, ch, cw],
                          axis=-1).reshape(h_n * w_n, _D)
    h_flat = grid[..., 0].reshape(-1)
    w_flat = grid[..., 1].reshape(-1)

    mesh = plsc.VectorSubcoreMesh(core_axis_name="c", subcore_axis_name="s")
    ker = pl.kernel(
        _sc_body,
        out_type=[jax.ShapeDtypeStruct((_N, 64), jnp.float32),
                  jax.ShapeDtypeStruct((_N, 64), jnp.float32)],
        mesh=mesh,
        scratch_types=[
            pltpu.VMEM((_CHUNK,), jnp.int32),
            pltpu.VMEM((_CHUNK,), jnp.int32),
            pltpu.VMEM((_NG, _GB), jnp.int32),
            pltpu.VMEM((_NB, _GB, _D), jnp.float32),
        ] + [pltpu.SemaphoreType.DMA] * (3 * _NB),
        compiler_params=pltpu.CompilerParams(use_tc_tiling_on_sc=False),
    )
    sin_flat, cos_flat = ker(h_flat, w_flat, tab)
    return (sin_flat.reshape(_B, _T, 64), cos_flat.reshape(_B, _T, 64))


def kernel(grid, cos_h_all, sin_h_all, cos_w_all, sin_w_all):
    return _rope_sc(grid, cos_h_all, sin_h_all, cos_w_all, sin_w_all)
